# baseline (speedup 1.0000x reference)
import jax
import jax.numpy as jnp
from jax import lax
from jax.experimental import pallas as pl
from jax.experimental.pallas import tpu as pltpu
from jax.experimental.pallas import tpu_sc as plsc

_NC = 2
_NS = 16
_NW = _NC * _NS
_LN = 16

_N = 10000
_NP = 10240
_D = 128
_DH = _D // _NC
_EV = 330000
_BS = 64
_NBUF = 4
_NSS = 3
_NR = 12
_C = 20736
_NB2 = _C // _BS
_EP = _NS * _C
_CA = _EP // _NW
_NBA = _CA // _LN

_SC_PARAMS = pltpu.CompilerParams(needs_layout_passes=False,
                                  use_tc_tiling_on_sc=False)
_SC_MESH = dict(mesh=plsc.VectorSubcoreMesh(core_axis_name="c",
                                            subcore_axis_name="s"),
                compiler_params=_SC_PARAMS)


def _proj_body(x_ref, w_ref, asrc_ref, adst_ref, h_ref, aa_ref):
    h = jnp.dot(x_ref[...], w_ref[...], preferred_element_type=jnp.float32)
    h_ref[0] = h[:, :_DH].astype(jnp.bfloat16)
    h_ref[1] = h[:, _DH:].astype(jnp.bfloat16)
    aa_ref[0, :] = jnp.sum(h * asrc_ref[...], axis=-1)
    aa_ref[1, :] = jnp.sum(h * adst_ref[...], axis=-1)


def _project(x, w, a_src, a_dst):
    bn = 1024
    h, aa = pl.pallas_call(
        _proj_body,
        grid=(_NP // bn,),
        in_specs=[
            pl.BlockSpec((bn, _D), lambda i: (i, 0)),
            pl.BlockSpec((_D, _D), lambda i: (0, 0)),
            pl.BlockSpec((1, _D), lambda i: (0, 0)),
            pl.BlockSpec((1, _D), lambda i: (0, 0)),
        ],
        out_specs=[
            pl.BlockSpec((_NC, bn, _DH), lambda i: (0, i, 0)),
            pl.BlockSpec((2, bn), lambda i: (0, i)),
        ],
        out_shape=[
            jax.ShapeDtypeStruct((_NC, _NP, _DH), jnp.bfloat16),
            jax.ShapeDtypeStruct((2, _NP), jnp.float32),
        ],
    )(x, w, a_src[None, :], a_dst[None, :])
    return h, aa


def _mid_body(acc_ref, dp_ref, b_ref, w_ref, asrc_ref, adst_ref,
              h_ref, aa_ref, den_ref):
    den = jnp.sum(dp_ref[...], axis=0)
    hm = jnp.concatenate([acc_ref[0], acc_ref[1]], axis=-1)
    hm = hm / den[:, None] + b_ref[...]
    hm = jnp.maximum(hm, 0.0)
    h = jnp.dot(hm, w_ref[...], preferred_element_type=jnp.float32)
    h_ref[0] = h[:, :_DH].astype(jnp.bfloat16)
    h_ref[1] = h[:, _DH:].astype(jnp.bfloat16)
    aa_ref[0, :] = jnp.sum(h * asrc_ref[...], axis=-1)
    aa_ref[1, :] = jnp.sum(h * adst_ref[...], axis=-1)
    den_ref[0, :] = den


def _mid(acc, dparts, b, w, a_src, a_dst):
    bn = 1024
    return pl.pallas_call(
        _mid_body,
        grid=(_NP // bn,),
        in_specs=[
            pl.BlockSpec((_NC, bn, _DH), lambda i: (0, i, 0)),
            pl.BlockSpec((_NW, bn), lambda i: (0, i)),
            pl.BlockSpec((1, _D), lambda i: (0, 0)),
            pl.BlockSpec((_D, _D), lambda i: (0, 0)),
            pl.BlockSpec((1, _D), lambda i: (0, 0)),
            pl.BlockSpec((1, _D), lambda i: (0, 0)),
        ],
        out_specs=[
            pl.BlockSpec((_NC, bn, _DH), lambda i: (0, i, 0)),
            pl.BlockSpec((2, bn), lambda i: (0, i)),
            pl.BlockSpec((1, bn), lambda i: (0, i)),
        ],
        out_shape=[
            jax.ShapeDtypeStruct((_NC, _NP, _DH), jnp.bfloat16),
            jax.ShapeDtypeStruct((2, _NP), jnp.float32),
            jax.ShapeDtypeStruct((1, _NP), jnp.float32),
        ],
    )(acc, dparts, b[None, :], w, a_src[None, :], a_dst[None, :])


def _fin_body(acc_ref, dp_ref, b_ref, o_ref):
    den = jnp.sum(dp_ref[...], axis=0)
    hm = jnp.concatenate([acc_ref[0], acc_ref[1]], axis=-1)
    o_ref[...] = hm / den[:, None] + b_ref[...]


def _final(acc, dparts, b):
    bn = 1024
    return pl.pallas_call(
        _fin_body,
        grid=(_NP // bn,),
        in_specs=[
            pl.BlockSpec((_NC, bn, _DH), lambda i: (0, i, 0)),
            pl.BlockSpec((_NW, bn), lambda i: (0, i)),
            pl.BlockSpec((1, _D), lambda i: (0, 0)),
        ],
        out_specs=pl.BlockSpec((bn, _D), lambda i: (i, 0)),
        out_shape=jax.ShapeDtypeStruct((_NP, _D), jnp.float32),
    )(acc, dparts, b[None, :])


def _att_body(asrc_hbm, adst_hbm, src3_hbm, dst3_hbm,
              ex_hbm, dp_hbm,
              as_v, ad_v, si_v, di_v, ex_v, den_v):
    cid = lax.axis_index("c")
    sid = lax.axis_index("s")
    wid = cid * _NS + sid
    base = wid * _CA

    pltpu.sync_copy(asrc_hbm, as_v)
    pltpu.sync_copy(adst_hbm, ad_v)
    pltpu.sync_copy(src3_hbm.at[wid], si_v)
    pltpu.sync_copy(dst3_hbm.at[wid], di_v)

    def zden(i, _):
        den_v[pl.ds(i * _LN, _LN)] = jnp.zeros((_LN,), jnp.float32)
        return 0
    lax.fori_loop(0, _NP // _LN, zden, 0)

    def p1(i, _):
        for u in range(2):
            iu = 2 * i + u
            sv = si_v[iu]
            dv = di_v[iu]
            e = plsc.load_gather(as_v, [sv]) + plsc.load_gather(ad_v, [dv])
            e = jnp.where(e >= 0.0, e, e * 0.2)
            ex = jnp.exp(e)
            gid = base + iu * _LN + lax.iota(jnp.int32, 16)
            ex = jnp.where(gid < _EV, ex, 0.0)
            ex_v[pl.ds(iu * _LN, _LN)] = ex
            plsc.addupdate_scatter(den_v, [dv], ex)
        return 0
    lax.fori_loop(0, _NBA // 2, p1, 0)

    pltpu.sync_copy(ex_v, ex_hbm.at[pl.ds(base, _CA)])
    pltpu.sync_copy(den_v, dp_hbm.at[wid])


_att_call = pl.kernel(
    _att_body,
    out_type=[
        jax.ShapeDtypeStruct((_EP,), jnp.float32),
        jax.ShapeDtypeStruct((_NW, _NP), jnp.float32),
    ],
    scratch_types=[
        pltpu.VMEM((_NP,), jnp.float32),
        pltpu.VMEM((_NP,), jnp.float32),
        pltpu.VMEM((_NBA, _LN), jnp.int32),
        pltpu.VMEM((_NBA, _LN), jnp.int32),
        pltpu.VMEM((_CA,), jnp.float32),
        pltpu.VMEM((_NP,), jnp.float32),
    ],
    **_SC_MESH,
)


def _acc_body(ex_hbm, sd_hbm, h_hbm, acc_hbm,
              sd_r, ex_r, rbufs, sbufs, gsems, ssems, isems, acc_sh):
    cid = lax.axis_index("c")
    sid = lax.axis_index("s")
    base = sid * _C
    sdc = sd_hbm.at[sid]
    hc = h_hbm.at[cid]

    for j in range(_LN):
        for s in range(_DH // _LN):
            sbufs[0][j, pl.ds(s * _LN, _LN)] = jnp.zeros((_LN,), jnp.float32)
    rows_per_tile = _NP // _NS
    for r in range(rows_per_tile // _LN):
        pltpu.sync_copy(sbufs[0].at[pl.ds(0, _LN)],
                        acc_sh.at[pl.ds(sid * rows_per_tile + r * _LN, _LN)])

    plsc.subcore_barrier()

    def fetch_idx(bb, r):
        pltpu.async_copy(sdc.at[bb], sd_r.at[r], isems[r])
        pltpu.async_copy(ex_hbm.at[pl.ds(base + bb * _BS, _BS)],
                         ex_r.at[r], isems[r])

    def wait_idx(r):
        pltpu.make_async_copy(sdc.at[0], sd_r.at[r], isems[r]).wait()
        pltpu.make_async_copy(ex_hbm.at[pl.ds(base, _BS)],
                              ex_r.at[r], isems[r]).wait()

    for r in range(2 * _NBUF):
        fetch_idx(r, r)
    for k in range(_NBUF):
        wait_idx(k)
        pltpu.async_copy(hc.at[sd_r.at[k].at[0]], rbufs[k], gsems[k])

    def scale(r, r_v, s_v):
        for half in range(_BS // _LN):
            exv = ex_r[r, pl.ds(half * _LN, _LN)]
            for j in range(_LN):
                w = jnp.broadcast_to(exv[j], (_LN,))
                row = half * _LN + j
                for s in range(_DH // (2 * _LN)):
                    ab = r_v[row, pl.ds(s * 2 * _LN, 2 * _LN)]
                    a, b2 = plsc.unpack(ab, format=plsc.PackFormat.INTERLEAVED)
                    s_v[row, pl.ds(s * 2 * _LN, _LN)] = a * w
                    s_v[row, pl.ds(s * 2 * _LN + _LN, _LN)] = b2 * w

    def p2(i, _):
        for k2 in range(_NR):
            b = i * _NR + k2
            k = k2 % _NBUF
            s3 = k2 % _NSS
            rg = (k2 + _NBUF) % _NR
            rf = (k2 + 2 * _NBUF) % _NR

            pltpu.make_async_copy(hc.at[sd_r.at[0].at[0]], rbufs[k],
                                  gsems[k]).wait()

            @pl.when(b >= _NSS)
            def _():
                pltpu.make_async_copy(sbufs[s3], acc_sh.at[sd_r.at[0].at[1]],
                                      ssems[s3]).wait()

            @pl.when(b + 2 * _NBUF < _NB2)
            def _():
                fetch_idx(b + 2 * _NBUF, rf)

            scale(k2, rbufs[k], sbufs[s3])
            pltpu.async_copy(sbufs[s3], acc_sh.at[sd_r.at[k2].at[1]],
                             ssems[s3], add=True)

            @pl.when(b + _NBUF < _NB2)
            def _():
                wait_idx(rg)
                pltpu.async_copy(hc.at[sd_r.at[rg].at[0]], rbufs[k], gsems[k])
        return 0
    lax.fori_loop(0, _NB2 // _NR, p2, 0)

    for s3 in range(_NSS):
        pltpu.make_async_copy(sbufs[s3], acc_sh.at[sd_r.at[0].at[1]],
                              ssems[s3]).wait()

    plsc.subcore_barrier()
    pltpu.sync_copy(acc_sh.at[pl.ds(sid * rows_per_tile, rows_per_tile)],
                    acc_hbm.at[cid].at[pl.ds(sid * rows_per_tile, rows_per_tile)])


_acc_call = pl.kernel(
    _acc_body,
    out_type=jax.ShapeDtypeStruct((_NC, _NP, _DH), jnp.float32),
    scratch_types=[
        pltpu.VMEM((_NR, 2, _BS), jnp.int32),
        pltpu.VMEM((_NR, _BS), jnp.float32),
        [pltpu.VMEM((_BS, _DH), jnp.bfloat16) for _ in range(_NBUF)],
        [pltpu.VMEM((_BS, _DH), jnp.float32) for _ in range(_NSS)],
        [pltpu.SemaphoreType.DMA for _ in range(_NBUF)],
        [pltpu.SemaphoreType.DMA for _ in range(_NSS)],
        [pltpu.SemaphoreType.DMA for _ in range(_NR)],
        pltpu.VMEM_SHARED((_NP, _DH), jnp.float32),
    ],
    **_SC_MESH,
)


def _att_alpha_body(asrc_hbm, adst_hbm, src3_hbm, dst3_hbm, ex1_hbm, den1_hbm,
                    ex_hbm, dp_hbm, alpha_hbm,
                    as_v, ad_v, si_v, di_v, ex_v, den_v, ex1_v):
    cid = lax.axis_index("c")
    sid = lax.axis_index("s")
    wid = cid * _NS + sid
    base = wid * _CA

    pltpu.sync_copy(asrc_hbm, as_v)
    pltpu.sync_copy(adst_hbm, ad_v)
    pltpu.sync_copy(src3_hbm.at[wid], si_v)
    pltpu.sync_copy(dst3_hbm.at[wid], di_v)
    pltpu.sync_copy(ex1_hbm.at[pl.ds(base, _CA)], ex1_v)

    def zden(i, _):
        den_v[pl.ds(i * _LN, _LN)] = jnp.zeros((_LN,), jnp.float32)
        return 0
    lax.fori_loop(0, _NP // _LN, zden, 0)

    def p1(i, _):
        for u in range(2):
            iu = 2 * i + u
            sv = si_v[iu]
            dv = di_v[iu]
            e = plsc.load_gather(as_v, [sv]) + plsc.load_gather(ad_v, [dv])
            e = jnp.where(e >= 0.0, e, e * 0.2)
            ex = jnp.exp(e)
            gid = base + iu * _LN + lax.iota(jnp.int32, 16)
            ex = jnp.where(gid < _EV, ex, 0.0)
            ex_v[pl.ds(iu * _LN, _LN)] = ex
            plsc.addupdate_scatter(den_v, [dv], ex)
        return 0
    lax.fori_loop(0, _NBA // 2, p1, 0)

    pltpu.sync_copy(ex_v, ex_hbm.at[pl.ds(base, _CA)])
    pltpu.sync_copy(den_v, dp_hbm.at[wid])

    pltpu.sync_copy(den1_hbm, den_v)

    def pa(i, _):
        for u in range(2):
            iu = 2 * i + u
            dg = plsc.load_gather(den_v, [di_v[iu]])
            sl = pl.ds(iu * _LN, _LN)
            ex1_v[sl] = ex1_v[sl] / (dg + 1e-16)
        return 0
    lax.fori_loop(0, _NBA // 2, pa, 0)
    pltpu.sync_copy(ex1_v, alpha_hbm.at[pl.ds(base, _CA)])


_att_alpha_call = pl.kernel(
    _att_alpha_body,
    out_type=[
        jax.ShapeDtypeStruct((_EP,), jnp.float32),
        jax.ShapeDtypeStruct((_NW, _NP), jnp.float32),
        jax.ShapeDtypeStruct((_EP,), jnp.float32),
    ],
    scratch_types=[
        pltpu.VMEM((_NP,), jnp.float32),
        pltpu.VMEM((_NP,), jnp.float32),
        pltpu.VMEM((_NBA, _LN), jnp.int32),
        pltpu.VMEM((_NBA, _LN), jnp.int32),
        pltpu.VMEM((_CA,), jnp.float32),
        pltpu.VMEM((_NP,), jnp.float32),
        pltpu.VMEM((_CA,), jnp.float32),
    ],
    **_SC_MESH,
)


def kernel(x, edge_index, W1, a_src1, a_dst1, b1, W2, a_src2, a_dst2, b2):
    n = x.shape[0]
    loop = jnp.arange(n, dtype=edge_index.dtype)
    src = jnp.concatenate([edge_index[0], loop])
    dst = jnp.concatenate([edge_index[1], loop])
    srcp = jnp.pad(src, (0, _EP - _EV)).reshape(_NS, _NB2, _BS)
    dstp = jnp.pad(dst, (0, _EP - _EV)).reshape(_NS, _NB2, _BS)
    sd = jnp.stack([srcp, dstp], axis=2)
    srcpa = srcp.reshape(_NW, _NBA, _LN)
    dstpa = dstp.reshape(_NW, _NBA, _LN)
    xp = jnp.pad(x, ((0, _NP - n), (0, 0)))

    def _interleave(h):
        return (h.reshape(_NC, _NP, _DH // 32, 2, 16)
                .transpose(0, 1, 2, 4, 3)
                .reshape(_NC, _NP, _DH))

    h1, aa1 = _project(xp, W1, a_src1, a_dst1)
    ex1, dp1 = _att_call(aa1[0], aa1[1], srcpa, dstpa)
    acc1 = _acc_call(ex1, sd, _interleave(h1))
    h2, aa2, den1 = _mid(acc1, dp1, b1, W2, a_src2, a_dst2)
    ex2, dp2, alpha1 = _att_alpha_call(aa2[0], aa2[1], srcpa, dstpa,
                                       ex1, den1[0])
    acc2 = _acc_call(ex2, sd, _interleave(h2))
    out2 = _final(acc2, dp2, b2)

    return ((jnp.stack([src, dst]), alpha1[:_EV]), out2[:n])
